# baseline (speedup 1.0000x reference)
import functools

import jax
import jax.numpy as jnp
from jax import lax
from jax.experimental import pallas as pl
from jax.experimental.pallas import tpu as pltpu
from jax.experimental.pallas import tpu_sc as plsc

B = 16384
D = 64
NT = 6

VOCABS = (2, 806, 935, 846, 411, 11)


def _pad32(v):
    return (v + 31) // 32 * 32


TOFF = []
_o = 0
for _v in VOCABS:
    TOFF.append(_o)
    _o += _pad32(_v)
TOT_ROWS = _o
TBL_ROWS128 = TOT_ROWS * 32 // 128

NB = B // 32
CHUNK = 8
WPE = NT * D // 2

_info = plsc.get_sparse_core_info()
_NC = _info.num_cores
_NS = _info.num_subcores

_mesh = plsc.VectorSubcoreMesh(core_axis_name="c", subcore_axis_name="s")


@functools.partial(
    pl.kernel,
    mesh=_mesh,
    compiler_params=pltpu.CompilerParams(use_tc_tiling_on_sc=True),
    out_type=jax.ShapeDtypeStruct((B // 2, 2 * WPE), jnp.int32),
    scratch_types=[
        pltpu.VMEM((TBL_ROWS128, 128), jnp.int32),
        pltpu.VMEM((NT * 8, 128), jnp.int32),
        pltpu.VMEM((CHUNK, 2 * WPE), jnp.int32),
        pltpu.SemaphoreType.DMA,
    ],
)
def _emb_kernel(pids_h, cate_h, cust_h, brand_h, camp_h, price_h,
                wpids_h, wcate_h, wcust_h, wbrand_h, wcamp_h, wprice_h,
                out_h, tbl, idx6, obuf, wsem):
    wid = lax.axis_index("s") * _NC + lax.axis_index("c")
    lo2 = pl.multiple_of(wid * (NB // 2), NB // 2)

    wrefs = (wpids_h, wcate_h, wcust_h, wbrand_h, wcamp_h, wprice_h)
    for t in range(NT):
        pltpu.sync_copy(wrefs[t], tbl.at[pl.ds(TOFF[t] * 32 // 128,
                                               wrefs[t].shape[0])])

    irefs = (pids_h, cate_h, cust_h, brand_h, camp_h, price_h)
    r8 = pl.multiple_of((wid // 2) * 8, 8)
    iofs = (wid % 2) * NB
    for t in range(NT):
        pltpu.sync_copy(irefs[t].at[pl.ds(r8, 8)], idx6.at[pl.ds(t * 8, 8)])

    def lookup(t, tblrow, j):
        w = (TOFF[t] + tblrow) >> 2
        cb = ((TOFF[t] + tblrow) & 3) * 32
        orow = j >> 1
        oc = (j & 1) * WPE + t * 32
        for k in range(2):
            obuf[orow, pl.ds(oc + k * 16, 16)] = tbl[w, pl.ds(cb + k * 16, 16)]

    def do_pair(i, primed):
        vs = [idx6[t * 8 + ((iofs + i * 16) >> 7),
                   pl.ds(lax.rem(iofs + i * 16, 128), 16)] for t in range(NT)]
        for half in range(2):
            if primed:
                pltpu.make_async_copy(
                    obuf.at[pl.ds(half * (CHUNK // 2), CHUNK // 2)],
                    out_h.at[pl.ds(lo2, CHUNK // 2)], wsem).wait()
            for j in range(CHUNK):
                for t in range(NT):
                    lookup(t, vs[t][half * CHUNK + j], half * CHUNK + j)
            pltpu.async_copy(
                obuf.at[pl.ds(half * (CHUNK // 2), CHUNK // 2)],
                out_h.at[pl.ds(lo2 + i * CHUNK + half * (CHUNK // 2),
                               CHUNK // 2)], wsem)

    do_pair(0, False)

    def outer(i, carry):
        do_pair(i, True)
        return carry

    lax.fori_loop(1, NB // 16, outer, 0)
    for half in range(2):
        pltpu.make_async_copy(obuf.at[pl.ds(half * (CHUNK // 2), CHUNK // 2)],
                              out_h.at[pl.ds(lo2, CHUNK // 2)], wsem).wait()


def _wpack(w, rows):
    v = w.shape[0]
    if rows != v:
        w = jnp.concatenate([w, jnp.zeros((rows - v, D), w.dtype)], axis=0)
    s = w.astype(jnp.bfloat16).reshape(rows, D // 2, 2)
    return lax.bitcast_convert_type(s, jnp.int32).reshape(-1, 128)


def kernel(cate, customer, brand, campaign, price, pids, id, W_cate,
           W_customer, W_brand, W_campaign, W_price, W_pids, W_id):
    shp = (B // 128, 128)
    out = _emb_kernel(
        pids.reshape(shp), cate.reshape(shp), customer.reshape(shp),
        brand.reshape(shp), campaign.reshape(shp), price.reshape(shp),
        _wpack(W_pids, _pad32(VOCABS[0])), _wpack(W_cate, _pad32(VOCABS[1])),
        _wpack(W_customer, _pad32(VOCABS[2])),
        _wpack(W_brand, _pad32(VOCABS[3])),
        _wpack(W_campaign, _pad32(VOCABS[4])),
        _wpack(W_price, _pad32(VOCABS[5])))
    vals = lax.bitcast_convert_type(out, jnp.bfloat16)
    return vals.astype(jnp.float32).reshape(B, NT, D)


# speedup vs baseline: 10.6884x; 10.6884x over previous
import functools

import jax
import jax.numpy as jnp
from jax import lax
from jax.experimental import pallas as pl
from jax.experimental.pallas import tpu as pltpu
from jax.experimental.pallas import tpu_sc as plsc

B = 16384
D = 64
NT = 6

VOCABS = (2, 806, 935, 846, 411, 11)


def _pad32(v):
    return (v + 31) // 32 * 32


TOFF = []
_o = 0
for _v in VOCABS:
    TOFF.append(_o)
    _o += _pad32(_v)
TOT_ROWS = _o
TBL_ROWS128 = TOT_ROWS * 32 // 128

NB = B // 32
CHUNK = 8
WPE = NT * D // 2

_info = plsc.get_sparse_core_info()
_NC = _info.num_cores
_NS = _info.num_subcores

_mesh = plsc.VectorSubcoreMesh(core_axis_name="c", subcore_axis_name="s")


@functools.partial(
    pl.kernel,
    mesh=_mesh,
    compiler_params=pltpu.CompilerParams(use_tc_tiling_on_sc=True),
    out_type=jax.ShapeDtypeStruct((B // 2, 2 * WPE), jnp.int32),
    scratch_types=[
        pltpu.VMEM((TBL_ROWS128, 128), jnp.int32),
        pltpu.VMEM((NT * 8, 128), jnp.int32),
        pltpu.VMEM((CHUNK, 2 * WPE), jnp.int32),
        pltpu.SemaphoreType.DMA,
    ],
)
def _emb_kernel(pids_h, cate_h, cust_h, brand_h, camp_h, price_h,
                wpids_h, wcate_h, wcust_h, wbrand_h, wcamp_h, wprice_h,
                out_h, tbl, idx6, obuf, wsem):
    wid = lax.axis_index("s") * _NC + lax.axis_index("c")
    lo2 = pl.multiple_of(wid * (NB // 2), NB // 2)

    wrefs = (wpids_h, wcate_h, wcust_h, wbrand_h, wcamp_h, wprice_h)
    for t in range(NT):
        pltpu.sync_copy(wrefs[t], tbl.at[pl.ds(TOFF[t] * 32 // 128,
                                               wrefs[t].shape[0])])

    irefs = (pids_h, cate_h, cust_h, brand_h, camp_h, price_h)
    r8 = pl.multiple_of((wid // 2) * 8, 8)
    iofs = (wid % 2) * NB
    for t in range(NT):
        pltpu.sync_copy(irefs[t].at[pl.ds(r8, 8)], idx6.at[pl.ds(t * 8, 8)])

    def lookup(t, tblrow, j):
        w = (TOFF[t] + tblrow) >> 2
        cb = ((TOFF[t] + tblrow) & 3) * 32
        orow = j >> 1
        oc = (j & 1) * WPE + t * 32
        for k in range(2):
            obuf[orow, pl.ds(oc + k * 16, 16)] = tbl[w, pl.ds(cb + k * 16, 16)]

    def do_pair(i, primed):
        vs = [idx6[t * 8 + ((iofs + i * 16) >> 7),
                   pl.ds(lax.rem(iofs + i * 16, 128), 16)] for t in range(NT)]
        for half in range(2):
            if primed:
                pltpu.make_async_copy(
                    obuf.at[pl.ds(half * (CHUNK // 2), CHUNK // 2)],
                    out_h.at[pl.ds(lo2, CHUNK // 2)], wsem).wait()
            for j in range(CHUNK):
                for t in range(NT):
                    lookup(t, vs[t][half * CHUNK + j], half * CHUNK + j)
            pltpu.async_copy(
                obuf.at[pl.ds(half * (CHUNK // 2), CHUNK // 2)],
                out_h.at[pl.ds(lo2 + i * CHUNK + half * (CHUNK // 2),
                               CHUNK // 2)], wsem)

    do_pair(0, False)

    def outer(i, carry):
        do_pair(i, True)
        return carry

    lax.fori_loop(1, NB // 16, outer, 0)
    for half in range(2):
        pltpu.make_async_copy(obuf.at[pl.ds(half * (CHUNK // 2), CHUNK // 2)],
                              out_h.at[pl.ds(lo2, CHUNK // 2)], wsem).wait()


def _wpack(w, rows):
    v = w.shape[0]
    if rows != v:
        w = jnp.concatenate([w, jnp.zeros((rows - v, D), w.dtype)], axis=0)
    s = w.astype(jnp.bfloat16).reshape(rows, 2, D // 2).transpose(0, 2, 1)
    return lax.bitcast_convert_type(s, jnp.int32).reshape(-1, 128)


def kernel(cate, customer, brand, campaign, price, pids, id, W_cate,
           W_customer, W_brand, W_campaign, W_price, W_pids, W_id):
    shp = (B // 128, 128)
    out = _emb_kernel(
        pids.reshape(shp), cate.reshape(shp), customer.reshape(shp),
        brand.reshape(shp), campaign.reshape(shp), price.reshape(shp),
        _wpack(W_pids, _pad32(VOCABS[0])), _wpack(W_cate, _pad32(VOCABS[1])),
        _wpack(W_customer, _pad32(VOCABS[2])),
        _wpack(W_brand, _pad32(VOCABS[3])),
        _wpack(W_campaign, _pad32(VOCABS[4])),
        _wpack(W_price, _pad32(VOCABS[5])))
    lo = lax.bitcast_convert_type(out << 16, jnp.float32)
    hi = lax.bitcast_convert_type(out & jnp.int32(-65536), jnp.float32)
    vals = jnp.concatenate(
        [lo.reshape(B // 2, 2 * NT, 1, D // 2),
         hi.reshape(B // 2, 2 * NT, 1, D // 2)], axis=2)
    return vals.reshape(B, NT, D)


# speedup vs baseline: 20.9766x; 1.9626x over previous
import functools

import jax
import jax.numpy as jnp
from jax import lax
from jax.experimental import pallas as pl
from jax.experimental.pallas import tpu as pltpu
from jax.experimental.pallas import tpu_sc as plsc

B = 16384
D = 64
NT = 6

VOCABS = (2, 806, 935, 846, 411, 11)


def _pad32(v):
    return (v + 31) // 32 * 32


TOFF = []
_o = 0
for _v in VOCABS:
    TOFF.append(_o)
    _o += _pad32(_v)
TOT_ROWS = _o
TBL_ROWS128 = TOT_ROWS * 32 // 128

NB = B // 32
CHUNK = 8
WPE = NT * D // 2

_info = plsc.get_sparse_core_info()
_NC = _info.num_cores
_NS = _info.num_subcores

_mesh = plsc.VectorSubcoreMesh(core_axis_name="c", subcore_axis_name="s")


@functools.partial(
    pl.kernel,
    mesh=_mesh,
    compiler_params=pltpu.CompilerParams(use_tc_tiling_on_sc=True),
    out_type=jax.ShapeDtypeStruct((B // 2, 2 * WPE), jnp.int32),
    scratch_types=[
        pltpu.VMEM((TBL_ROWS128, 128), jnp.int32),
        pltpu.VMEM((NT * 8, 128), jnp.int32),
        pltpu.VMEM((CHUNK, 2 * WPE), jnp.int32),
        pltpu.SemaphoreType.DMA,
    ],
)
def _emb_kernel(pids_h, cate_h, cust_h, brand_h, camp_h, price_h,
                wpids_h, wcate_h, wcust_h, wbrand_h, wcamp_h, wprice_h,
                out_h, tbl, idx6, obuf, wsem):
    wid = lax.axis_index("s") * _NC + lax.axis_index("c")
    lo2 = pl.multiple_of(wid * (NB // 2), NB // 2)

    wrefs = (wpids_h, wcate_h, wcust_h, wbrand_h, wcamp_h, wprice_h)
    for t in range(NT):
        pltpu.sync_copy(wrefs[t], tbl.at[pl.ds(TOFF[t] * 32 // 128,
                                               wrefs[t].shape[0])])

    irefs = (pids_h, cate_h, cust_h, brand_h, camp_h, price_h)
    r8 = pl.multiple_of((wid // 2) * 8, 8)
    iofs = (wid % 2) * NB
    for t in range(NT):
        pltpu.sync_copy(irefs[t].at[pl.ds(r8, 8)], idx6.at[pl.ds(t * 8, 8)])

    def lookup(t, tblrow, j):
        w = (TOFF[t] + tblrow) >> 2
        cb = ((TOFF[t] + tblrow) & 3) * 32
        orow = j >> 1
        oc = (j & 1) * WPE + t * 32
        for k in range(2):
            obuf[orow, pl.ds(oc + k * 16, 16)] = tbl[w, pl.ds(cb + k * 16, 16)]

    def do_pair(i, primed):
        vs = [idx6[t * 8 + ((iofs + i * 16) >> 7),
                   pl.ds(lax.rem(iofs + i * 16, 128), 16)] for t in range(NT)]
        for half in range(2):
            if primed:
                pltpu.make_async_copy(
                    obuf.at[pl.ds(half * (CHUNK // 2), CHUNK // 2)],
                    out_h.at[pl.ds(lo2, CHUNK // 2)], wsem).wait()
            for j in range(CHUNK):
                for t in range(NT):
                    lookup(t, vs[t][half * CHUNK + j], half * CHUNK + j)
            pltpu.async_copy(
                obuf.at[pl.ds(half * (CHUNK // 2), CHUNK // 2)],
                out_h.at[pl.ds(lo2 + i * CHUNK + half * (CHUNK // 2),
                               CHUNK // 2)], wsem)

    do_pair(0, False)

    def outer(i, carry):
        do_pair(i, True)
        return carry

    lax.fori_loop(1, NB // 16, outer, 0)
    for half in range(2):
        pltpu.make_async_copy(obuf.at[pl.ds(half * (CHUNK // 2), CHUNK // 2)],
                              out_h.at[pl.ds(lo2, CHUNK // 2)], wsem).wait()


def _wpack(w, rows):
    v = w.shape[0]
    if rows != v:
        w = jnp.concatenate([w, jnp.zeros((rows - v, D), w.dtype)], axis=0)
    s = w.astype(jnp.bfloat16).reshape(rows, 2, D // 2).transpose(0, 2, 1)
    return lax.bitcast_convert_type(s, jnp.int32).reshape(-1, 128)


def kernel(cate, customer, brand, campaign, price, pids, id, W_cate,
           W_customer, W_brand, W_campaign, W_price, W_pids, W_id):
    shp = (B // 128, 128)
    out = _emb_kernel(
        pids.reshape(shp), cate.reshape(shp), customer.reshape(shp),
        brand.reshape(shp), campaign.reshape(shp), price.reshape(shp),
        _wpack(W_pids, _pad32(VOCABS[0])), _wpack(W_cate, _pad32(VOCABS[1])),
        _wpack(W_customer, _pad32(VOCABS[2])),
        _wpack(W_brand, _pad32(VOCABS[3])),
        _wpack(W_campaign, _pad32(VOCABS[4])),
        _wpack(W_price, _pad32(VOCABS[5])))
    out2 = out.reshape(B, WPE)
    lo = lax.bitcast_convert_type(out2 << 16, jnp.float32)
    hi = lax.bitcast_convert_type(out2 & jnp.int32(-65536), jnp.float32)
    return jnp.concatenate([lo.reshape(B, NT, D // 2),
                            hi.reshape(B, NT, D // 2)], axis=2)
